# baseline (speedup 1.0000x reference)
import functools

import jax
import jax.numpy as jnp
from jax.experimental import pallas as pl
from jax.experimental.pallas import tpu as pltpu

_N = 16384
_IN = 256
_HID = 128
_FEAT = 64
_K = 1024
_BLK = 512
_GRID = _N // _BLK


def _vq_kernel(x_ref, w1_ref, b1_ref, w2_ref, b2_ref, cb_ref, g_ref,
               nd_ref, qst_ref, enc_ref, loss_ref, perp_ref,
               counts_ref, ssum_ref):
    i = pl.program_id(0)

    @pl.when(i == 0)
    def _init():
        counts_ref[...] = jnp.zeros_like(counts_ref)
        ssum_ref[...] = jnp.zeros_like(ssum_ref)

    x = x_ref[...]
    w1 = w1_ref[...]
    w2 = w2_ref[...]
    cb = cb_ref[...]

    h = jax.lax.dot_general(x, w1, (((1,), (1,)), ((), ())),
                            preferred_element_type=jnp.float32)
    h = jnp.maximum(h + b1_ref[...], 0.0)
    z = jax.lax.dot_general(h, w2, (((1,), (1,)), ((), ())),
                            preferred_element_type=jnp.float32)
    z = z + b2_ref[...]

    zsq = jnp.sum(z * z, axis=1, keepdims=True)
    ones = jnp.ones((1, _FEAT), dtype=jnp.float32)
    cbsq = jax.lax.dot_general(ones, cb * cb, (((1,), (1,)), ((), ())),
                               preferred_element_type=jnp.float32)
    d2 = jax.lax.dot_general(z, cb, (((1,), (1,)), ((), ())),
                             preferred_element_type=jnp.float32)
    dist = (zsq + cbsq) - 2.0 * d2
    nd = -dist
    nd_ref[...] = nd

    scaled = nd / 0.1
    rowmax = jnp.max(scaled, axis=-1, keepdims=True)
    y = scaled - rowmax
    y = jnp.clip(y, -1000.0, 10.0)
    e = jnp.exp(y)
    s = jnp.sum(e, axis=-1, keepdims=True)
    p = e / s

    v = g_ref[...] + jnp.log(p + 1e-20)
    idx = jnp.argmax(v, axis=-1, keepdims=True)
    enc = (jax.lax.broadcasted_iota(jnp.int32, (_BLK, _K), 1)
           == idx).astype(jnp.float32)
    enc_ref[...] = enc

    q = jnp.dot(enc, cb, preferred_element_type=jnp.float32)
    diff = q - z
    qst_ref[...] = z + diff

    ones_rows = jnp.ones((1, _BLK), dtype=jnp.float32)
    counts_ref[...] += jnp.dot(ones_rows, enc,
                               preferred_element_type=jnp.float32)
    ssum_ref[...] += jnp.sum(diff * diff).reshape(1, 1)

    @pl.when(i == _GRID - 1)
    def _fini():
        m = ssum_ref[...] / jnp.float32(_N * _FEAT)
        loss_ref[...] = m + 1.0 * m
        avg = counts_ref[...] / jnp.float32(_N)
        ent = jnp.sum(avg * jnp.log(avg + 1e-10)).reshape(1, 1)
        perp_ref[...] = jnp.exp(-ent)


@functools.partial(jax.jit, static_argnames=())
def _run(input_data, W1, b1, W2, b2, code_book, gumbel):
    grid_spec = pltpu.PrefetchScalarGridSpec(
        num_scalar_prefetch=0,
        grid=(_GRID,),
        in_specs=[
            pl.BlockSpec((_BLK, _IN), lambda i: (i, 0)),
            pl.BlockSpec((_HID, _IN), lambda i: (0, 0)),
            pl.BlockSpec((1, _HID), lambda i: (0, 0)),
            pl.BlockSpec((_FEAT, _HID), lambda i: (0, 0)),
            pl.BlockSpec((1, _FEAT), lambda i: (0, 0)),
            pl.BlockSpec((_K, _FEAT), lambda i: (0, 0)),
            pl.BlockSpec((_BLK, _K), lambda i: (i, 0)),
        ],
        out_specs=[
            pl.BlockSpec((_BLK, _K), lambda i: (i, 0)),
            pl.BlockSpec((_BLK, _FEAT), lambda i: (i, 0)),
            pl.BlockSpec((_BLK, _K), lambda i: (i, 0)),
            pl.BlockSpec((1, 1), lambda i: (0, 0)),
            pl.BlockSpec((1, 1), lambda i: (0, 0)),
        ],
        scratch_shapes=[
            pltpu.VMEM((1, _K), jnp.float32),
            pltpu.VMEM((1, 1), jnp.float32),
        ],
    )
    nd, qst, enc, loss, perp = pl.pallas_call(
        _vq_kernel,
        grid_spec=grid_spec,
        out_shape=[
            jax.ShapeDtypeStruct((_N, _K), jnp.float32),
            jax.ShapeDtypeStruct((_N, _FEAT), jnp.float32),
            jax.ShapeDtypeStruct((_N, _K), jnp.float32),
            jax.ShapeDtypeStruct((1, 1), jnp.float32),
            jax.ShapeDtypeStruct((1, 1), jnp.float32),
        ],
        compiler_params=pltpu.CompilerParams(
            dimension_semantics=("arbitrary",),
        ),
    )(input_data, W1, b1.reshape(1, _HID), W2, b2.reshape(1, _FEAT),
      code_book, gumbel)
    return (loss.reshape(()), qst, perp.reshape(()), enc, nd)


@functools.lru_cache(maxsize=1)
def _gumbel_const():
    with jax.ensure_compile_time_eval():
        return jax.random.gumbel(jax.random.key(42), (_N, _K), jnp.float32)


def kernel(input_data, W1, b1, W2, b2, code_book):
    return _run(input_data, W1, b1, W2, b2, code_book, _gumbel_const())


# speedup vs baseline: 1.0928x; 1.0928x over previous
import functools

import jax
import jax.numpy as jnp
from jax.experimental import pallas as pl
from jax.experimental.pallas import tpu as pltpu

_N = 16384
_IN = 256
_HID = 128
_FEAT = 64
_K = 1024
_BLK = 1024
_GRID = _N // _BLK


def _vq_kernel(x_ref, w1_ref, b1_ref, w2_ref, b2_ref, cb_ref, g_ref,
               nd_ref, qst_ref, enc_ref, loss_ref, perp_ref,
               counts_ref, ssum_ref):
    i = pl.program_id(0)

    @pl.when(i == 0)
    def _init():
        counts_ref[...] = jnp.zeros_like(counts_ref)
        ssum_ref[...] = jnp.zeros_like(ssum_ref)

    x = x_ref[...]
    w1 = w1_ref[...]
    w2 = w2_ref[...]
    cb = cb_ref[...]

    h = jax.lax.dot_general(x, w1, (((1,), (1,)), ((), ())),
                            preferred_element_type=jnp.float32)
    h = jnp.maximum(h + b1_ref[...], 0.0)
    z = jax.lax.dot_general(h, w2, (((1,), (1,)), ((), ())),
                            preferred_element_type=jnp.float32)
    z = z + b2_ref[...]

    zsq = jnp.sum(z * z, axis=1, keepdims=True)
    ones = jnp.ones((1, _FEAT), dtype=jnp.float32)
    cbsq = jax.lax.dot_general(ones, cb * cb, (((1,), (1,)), ((), ())),
                               preferred_element_type=jnp.float32)
    d2 = jax.lax.dot_general(z, cb, (((1,), (1,)), ((), ())),
                             preferred_element_type=jnp.float32)
    dist = (zsq + cbsq) - 2.0 * d2
    nd = -dist
    nd_ref[...] = nd

    scaled = nd / 0.1
    rowmax = jnp.max(scaled, axis=-1, keepdims=True)
    y = scaled - rowmax
    y = jnp.clip(y, -1000.0, 10.0)
    e = jnp.exp(y)
    s = jnp.sum(e, axis=-1, keepdims=True)
    p = e / s

    v = g_ref[...] + jnp.log(p + 1e-20)
    idx = jnp.argmax(v, axis=-1, keepdims=True)
    enc = (jax.lax.broadcasted_iota(jnp.int32, (_BLK, _K), 1)
           == idx).astype(jnp.float32)
    enc_ref[...] = enc

    q = jnp.dot(enc, cb, preferred_element_type=jnp.float32)
    diff = q - z
    qst_ref[...] = z + diff

    ones_rows = jnp.ones((1, _BLK), dtype=jnp.float32)
    counts_ref[...] += jnp.dot(ones_rows, enc,
                               preferred_element_type=jnp.float32)
    ssum_ref[...] += jnp.sum(diff * diff).reshape(1, 1)

    @pl.when(i == _GRID - 1)
    def _fini():
        m = ssum_ref[...] / jnp.float32(_N * _FEAT)
        loss_ref[...] = m + 1.0 * m
        avg = counts_ref[...] / jnp.float32(_N)
        ent = jnp.sum(avg * jnp.log(avg + 1e-10)).reshape(1, 1)
        perp_ref[...] = jnp.exp(-ent)


@functools.partial(jax.jit, static_argnames=())
def _run(input_data, W1, b1, W2, b2, code_book, gumbel):
    grid_spec = pltpu.PrefetchScalarGridSpec(
        num_scalar_prefetch=0,
        grid=(_GRID,),
        in_specs=[
            pl.BlockSpec((_BLK, _IN), lambda i: (i, 0)),
            pl.BlockSpec((_HID, _IN), lambda i: (0, 0)),
            pl.BlockSpec((1, _HID), lambda i: (0, 0)),
            pl.BlockSpec((_FEAT, _HID), lambda i: (0, 0)),
            pl.BlockSpec((1, _FEAT), lambda i: (0, 0)),
            pl.BlockSpec((_K, _FEAT), lambda i: (0, 0)),
            pl.BlockSpec((_BLK, _K), lambda i: (i, 0)),
        ],
        out_specs=[
            pl.BlockSpec((_BLK, _K), lambda i: (i, 0)),
            pl.BlockSpec((_BLK, _FEAT), lambda i: (i, 0)),
            pl.BlockSpec((_BLK, _K), lambda i: (i, 0)),
            pl.BlockSpec((1, 1), lambda i: (0, 0)),
            pl.BlockSpec((1, 1), lambda i: (0, 0)),
        ],
        scratch_shapes=[
            pltpu.VMEM((1, _K), jnp.float32),
            pltpu.VMEM((1, 1), jnp.float32),
        ],
    )
    nd, qst, enc, loss, perp = pl.pallas_call(
        _vq_kernel,
        grid_spec=grid_spec,
        out_shape=[
            jax.ShapeDtypeStruct((_N, _K), jnp.float32),
            jax.ShapeDtypeStruct((_N, _FEAT), jnp.float32),
            jax.ShapeDtypeStruct((_N, _K), jnp.float32),
            jax.ShapeDtypeStruct((1, 1), jnp.float32),
            jax.ShapeDtypeStruct((1, 1), jnp.float32),
        ],
        compiler_params=pltpu.CompilerParams(
            dimension_semantics=("arbitrary",),
        ),
    )(input_data, W1, b1.reshape(1, _HID), W2, b2.reshape(1, _FEAT),
      code_book, gumbel)
    return (loss.reshape(()), qst, perp.reshape(()), enc, nd)


@functools.lru_cache(maxsize=1)
def _gumbel_const():
    with jax.ensure_compile_time_eval():
        return jax.random.gumbel(jax.random.key(42), (_N, _K), jnp.float32)


def kernel(input_data, W1, b1, W2, b2, code_book):
    return _run(input_data, W1, b1, W2, b2, code_book, _gumbel_const())
